# baseline (speedup 1.0000x reference)
import jax
import jax.numpy as jnp
from jax import lax
from jax.experimental import pallas as pl
from jax.experimental.pallas import tpu as pltpu
from jax.experimental.pallas import tpu_sc as plsc

N = 10000
E = 320000
D = 128
DH = D // 2
OUT = 121

NC, NS = 2, 16
CH = 128
NP = 10240
STRIPE = NP // NS
KBUF = 3
NCHUNK = KBUF * (-(-E // (NS * CH * KBUF)))
EPAD = NS * CH * NCHUNK


def _sc_agg(with_cnt):
    mesh = plsc.VectorSubcoreMesh(core_axis_name="c", subcore_axis_name="s")
    out_type = [jax.ShapeDtypeStruct((NC, NP, DH), jnp.float32)]
    scratch = [
        pltpu.VMEM((2, NCHUNK, CH), jnp.int32),
    ] + [pltpu.VMEM((CH, DH), jnp.float32) for _ in range(KBUF)] + [
        pltpu.VMEM((CH, DH), jnp.float32),
        pltpu.VMEM_SHARED((NP, DH), jnp.float32),
    ] + [pltpu.SemaphoreType.DMA for _ in range(KBUF)]
    if with_cnt:
        out_type.append(jax.ShapeDtypeStruct((NC, NP), jnp.float32))
        scratch += [
            pltpu.VMEM((CH,), jnp.float32),
            pltpu.VMEM((STRIPE,), jnp.float32),
            pltpu.VMEM_SHARED((NP,), jnp.float32),
            pltpu.SemaphoreType.DMA,
        ]

    def body(x2_hbm, edges_hbm, *rest):
        if with_cnt:
            sums_out, cnt_out = rest[0], rest[1]
            rest = rest[2:]
        else:
            sums_out = rest[0]
            rest = rest[1:]
        idx_v = rest[0]
        bufs = rest[1:1 + KBUF]
        zero_v = rest[1 + KBUF]
        sums_sh = rest[2 + KBUF]
        gsems = rest[3 + KBUF:3 + 2 * KBUF]
        if with_cnt:
            ones_v, zcnt_v, cnt_sh, csem = rest[3 + 2 * KBUF:]
        cid = lax.axis_index("c")
        sid = lax.axis_index("s")

        pltpu.sync_copy(edges_hbm.at[sid], idx_v)
        src_v = idx_v.at[0]
        dst_v = idx_v.at[1]

        @pl.loop(0, CH)
        def _zero_rows(i):
            for j in range(DH // 16):
                zero_v[i, pl.ds(j * 16, 16)] = jnp.zeros((16,), jnp.float32)

        for k in range(STRIPE // CH):
            pltpu.sync_copy(zero_v, sums_sh.at[pl.ds(sid * STRIPE + k * CH, CH)])
        if with_cnt:
            @pl.loop(0, STRIPE // 16)
            def _zero_cnt(i):
                zcnt_v[pl.ds(i * 16, 16)] = jnp.zeros((16,), jnp.float32)

            @pl.loop(0, CH // 16)
            def _ones(i):
                ones_v[pl.ds(i * 16, 16)] = jnp.ones((16,), jnp.float32)

            pltpu.sync_copy(zcnt_v, cnt_sh.at[pl.ds(sid * STRIPE, STRIPE)])
        plsc.subcore_barrier()

        cnt_lo = cid * (NCHUNK // 2)
        cnt_hi = cnt_lo + NCHUNK // 2

        def _maybe_cnt(j):
            if with_cnt:
                @pl.when((j >= cnt_lo) & (j < cnt_hi))
                def _():
                    pltpu.async_copy(ones_v, cnt_sh.at[dst_v.at[j]], csem,
                                     add=True)

        for b in range(KBUF):
            pltpu.async_copy(x2_hbm.at[cid].at[src_v.at[b]], bufs[b], gsems[b])

        @pl.loop(0, NCHUNK, step=KBUF)
        def _chunk(j):
            for b in range(KBUF):
                jj = j + b
                pltpu.make_async_copy(x2_hbm.at[cid].at[src_v.at[jj]],
                                      bufs[b], gsems[b]).wait()
                pltpu.sync_copy(bufs[b], sums_sh.at[dst_v.at[jj]], add=True)
                _maybe_cnt(jj)

                @pl.when(jj + KBUF < NCHUNK)
                def _():
                    pltpu.async_copy(x2_hbm.at[cid].at[src_v.at[jj + KBUF]],
                                     bufs[b], gsems[b])

        if with_cnt:
            @pl.loop(cnt_lo, cnt_hi)
            def _drain_cnt(j):
                pltpu.make_async_copy(ones_v, cnt_sh.at[dst_v.at[j]],
                                      csem).wait()
        plsc.subcore_barrier()
        pltpu.sync_copy(sums_sh.at[pl.ds(sid * STRIPE, STRIPE)],
                        sums_out.at[cid, pl.ds(sid * STRIPE, STRIPE)])
        if with_cnt:
            pltpu.sync_copy(cnt_sh.at[pl.ds(sid * STRIPE, STRIPE)],
                            cnt_out.at[cid, pl.ds(sid * STRIPE, STRIPE)])

    return pl.kernel(body, out_type=out_type, mesh=mesh, scratch_types=scratch,
                     compiler_params=pltpu.CompilerParams(use_tc_tiling_on_sc=False),
                     name="sc_agg_cnt" if with_cnt else "sc_agg")


_sc_agg_cnt_kernel = _sc_agg(True)
_sc_agg_kernel = _sc_agg(False)


def _tc_layer(relu, split_out):
    TR = 1024 if split_out else 1000

    def body(sums_ref, cnt_ref, x_ref, wl_ref, wr_ref, b_ref, o_ref):
        c = cnt_ref[0] + cnt_ref[1]
        s = jnp.concatenate([sums_ref[0], sums_ref[1]], axis=1)
        mean = s * (1.0 / jnp.maximum(c, 1.0))
        xcat = jnp.concatenate([x_ref[0], x_ref[1]], axis=1)
        acc = jnp.dot(mean, wl_ref[...], preferred_element_type=jnp.float32)
        acc = acc + jnp.dot(xcat, wr_ref[...],
                            preferred_element_type=jnp.float32)
        acc = acc + b_ref[...]
        if relu:
            acc = jnp.maximum(acc, 0.0)
        if split_out:
            o_ref[0] = acc[:, :DH]
            o_ref[1] = acc[:, DH:]
        else:
            o_ref[...] = acc[:, :OUT]

    if split_out:
        out_specs = pl.BlockSpec((NC, TR, DH), lambda i: (0, i, 0))
        out_shape = jax.ShapeDtypeStruct((NC, NP, DH), jnp.float32)
    else:
        out_specs = pl.BlockSpec((TR, OUT), lambda i: (i, 0))
        out_shape = jax.ShapeDtypeStruct((N, OUT), jnp.float32)
    return pl.pallas_call(
        body,
        grid=((NP // TR) if split_out else (N // TR),),
        in_specs=[
            pl.BlockSpec((NC, TR, DH), lambda i: (0, i, 0)),
            pl.BlockSpec((NC, TR, 1), lambda i: (0, i, 0)),
            pl.BlockSpec((NC, TR, DH), lambda i: (0, i, 0)),
            pl.BlockSpec((D, D), lambda i: (0, 0)),
            pl.BlockSpec((D, D), lambda i: (0, 0)),
            pl.BlockSpec((1, D), lambda i: (0, 0)),
        ],
        out_specs=out_specs,
        out_shape=out_shape,
        name="tc_layer_relu" if relu else "tc_layer",
    )


_tc_layer_relu = _tc_layer(True, True)
_tc_layer_lin = _tc_layer(False, False)


def kernel(x, edge_index, W1l, b1l, W1r, b1r, W2l, b2l, W2r, b2r):
    src = edge_index[0]
    dst = edge_index[1]
    pad = EPAD - E
    srcs = jnp.concatenate([src, jnp.zeros((pad,), jnp.int32)]).reshape(NS, NCHUNK, CH)
    dsts = jnp.concatenate([dst, jnp.full((pad,), N, jnp.int32)]).reshape(NS, NCHUNK, CH)
    edges = jnp.stack([srcs, dsts], axis=1)

    xp = jnp.zeros((NP, D), jnp.float32).at[:N].set(x)
    x2 = jnp.stack([xp[:, :DH], xp[:, DH:]])
    w1l_t = W1l.T
    w1r_t = W1r.T
    b1 = (b1l + b1r).reshape(1, D)
    w2l_t = jnp.zeros((D, D), jnp.float32).at[:, :OUT].set(W2l.T)
    w2r_t = jnp.zeros((D, D), jnp.float32).at[:, :OUT].set(W2r.T)
    b2 = jnp.zeros((1, D), jnp.float32).at[0, :OUT].set(b2l + b2r)

    sums1, cnt = _sc_agg_cnt_kernel(x2, edges)
    cnt3 = cnt.reshape(NC, NP, 1)
    h2 = _tc_layer_relu(sums1, cnt3, x2, w1l_t, w1r_t, b1)
    (sums2,) = _sc_agg_kernel(h2, edges)
    return _tc_layer_lin(sums2, cnt3, h2, w2l_t, w2r_t, b2)


# speedup vs baseline: 1.0191x; 1.0191x over previous
import jax
import jax.numpy as jnp
from jax import lax
from jax.experimental import pallas as pl
from jax.experimental.pallas import tpu as pltpu
from jax.experimental.pallas import tpu_sc as plsc

N = 10000
E = 320000
D = 128
DH = D // 2
OUT = 121

NC, NS = 2, 16
CH = 128
NP = 10240
STRIPE = NP // NS
KBUF = 2
NCHUNK = KBUF * (-(-E // (NS * CH * KBUF)))
EPAD = NS * CH * NCHUNK


def _sc_agg(with_cnt):
    mesh = plsc.VectorSubcoreMesh(core_axis_name="c", subcore_axis_name="s")
    out_type = [jax.ShapeDtypeStruct((NC, NP, DH), jnp.float32)]
    scratch = [
        pltpu.VMEM((2, NCHUNK, CH), jnp.int32),
    ] + [pltpu.VMEM((CH, DH), jnp.float32) for _ in range(KBUF)] + [
        pltpu.VMEM((CH, DH), jnp.float32),
        pltpu.VMEM_SHARED((NP, DH), jnp.float32),
    ] + [pltpu.SemaphoreType.DMA for _ in range(KBUF)]
    if with_cnt:
        out_type.append(jax.ShapeDtypeStruct((NC, NP), jnp.float32))
        scratch += [
            pltpu.VMEM((CH,), jnp.float32),
            pltpu.VMEM((STRIPE,), jnp.float32),
            pltpu.VMEM_SHARED((NP,), jnp.float32),
            pltpu.SemaphoreType.DMA,
        ]

    def body(x2_hbm, edges_hbm, *rest):
        if with_cnt:
            sums_out, cnt_out = rest[0], rest[1]
            rest = rest[2:]
        else:
            sums_out = rest[0]
            rest = rest[1:]
        idx_v = rest[0]
        bufs = rest[1:1 + KBUF]
        zero_v = rest[1 + KBUF]
        sums_sh = rest[2 + KBUF]
        gsems = rest[3 + KBUF:3 + 2 * KBUF]
        if with_cnt:
            ones_v, zcnt_v, cnt_sh, csem = rest[3 + 2 * KBUF:]
        cid = lax.axis_index("c")
        sid = lax.axis_index("s")

        pltpu.sync_copy(edges_hbm.at[sid], idx_v)
        src_v = idx_v.at[0]
        dst_v = idx_v.at[1]

        @pl.loop(0, CH)
        def _zero_rows(i):
            for j in range(DH // 16):
                zero_v[i, pl.ds(j * 16, 16)] = jnp.zeros((16,), jnp.float32)

        for k in range(STRIPE // CH):
            pltpu.sync_copy(zero_v, sums_sh.at[pl.ds(sid * STRIPE + k * CH, CH)])
        if with_cnt:
            @pl.loop(0, STRIPE // 16)
            def _zero_cnt(i):
                zcnt_v[pl.ds(i * 16, 16)] = jnp.zeros((16,), jnp.float32)

            @pl.loop(0, CH // 16)
            def _ones(i):
                ones_v[pl.ds(i * 16, 16)] = jnp.ones((16,), jnp.float32)

            pltpu.sync_copy(zcnt_v, cnt_sh.at[pl.ds(sid * STRIPE, STRIPE)])
        plsc.subcore_barrier()

        cnt_lo = cid * (NCHUNK // 2)
        cnt_hi = cnt_lo + NCHUNK // 2

        def _maybe_cnt(j):
            if with_cnt:
                @pl.when((j >= cnt_lo) & (j < cnt_hi))
                def _():
                    pltpu.async_copy(ones_v, cnt_sh.at[dst_v.at[j]], csem,
                                     add=True)

        for b in range(KBUF):
            pltpu.async_copy(x2_hbm.at[cid].at[src_v.at[b]], bufs[b], gsems[b])

        @pl.loop(0, NCHUNK, step=KBUF)
        def _chunk(j):
            for b in range(KBUF):
                jj = j + b
                pltpu.make_async_copy(x2_hbm.at[cid].at[src_v.at[jj]],
                                      bufs[b], gsems[b]).wait()
                pltpu.sync_copy(bufs[b], sums_sh.at[dst_v.at[jj]], add=True)
                _maybe_cnt(jj)

                @pl.when(jj + KBUF < NCHUNK)
                def _():
                    pltpu.async_copy(x2_hbm.at[cid].at[src_v.at[jj + KBUF]],
                                     bufs[b], gsems[b])

        if with_cnt:
            @pl.loop(cnt_lo, cnt_hi)
            def _drain_cnt(j):
                pltpu.make_async_copy(ones_v, cnt_sh.at[dst_v.at[j]],
                                      csem).wait()
        plsc.subcore_barrier()
        pltpu.sync_copy(sums_sh.at[pl.ds(sid * STRIPE, STRIPE)],
                        sums_out.at[cid, pl.ds(sid * STRIPE, STRIPE)])
        if with_cnt:
            pltpu.sync_copy(cnt_sh.at[pl.ds(sid * STRIPE, STRIPE)],
                            cnt_out.at[cid, pl.ds(sid * STRIPE, STRIPE)])

    return pl.kernel(body, out_type=out_type, mesh=mesh, scratch_types=scratch,
                     compiler_params=pltpu.CompilerParams(use_tc_tiling_on_sc=False),
                     name="sc_agg_cnt" if with_cnt else "sc_agg")


_sc_agg_cnt_kernel = _sc_agg(True)
_sc_agg_kernel = _sc_agg(False)


def _tc_layer(relu, split_out):
    TR = 1000

    def body(sums_ref, cnt_ref, x_ref, wl_ref, wr_ref, b_ref, o_ref):
        c = cnt_ref[0] + cnt_ref[1]
        s = jnp.concatenate([sums_ref[0], sums_ref[1]], axis=1)
        mean = s * (1.0 / jnp.maximum(c, 1.0))
        xcat = jnp.concatenate([x_ref[0], x_ref[1]], axis=1)
        acc = jnp.dot(mean, wl_ref[...], preferred_element_type=jnp.float32)
        acc = acc + jnp.dot(xcat, wr_ref[...],
                            preferred_element_type=jnp.float32)
        acc = acc + b_ref[...]
        if relu:
            acc = jnp.maximum(acc, 0.0)
        if split_out:
            o_ref[0] = acc[:, :DH]
            o_ref[1] = acc[:, DH:]
        else:
            o_ref[...] = acc[:, :OUT]

    if split_out:
        out_specs = pl.BlockSpec((NC, TR, DH), lambda i: (0, i, 0))
        out_shape = jax.ShapeDtypeStruct((NC, N, DH), jnp.float32)
    else:
        out_specs = pl.BlockSpec((TR, OUT), lambda i: (i, 0))
        out_shape = jax.ShapeDtypeStruct((N, OUT), jnp.float32)
    return pl.pallas_call(
        body,
        grid=(N // TR,),
        in_specs=[
            pl.BlockSpec((NC, TR, DH), lambda i: (0, i, 0)),
            pl.BlockSpec((NC, TR, 1), lambda i: (0, i, 0)),
            pl.BlockSpec((NC, TR, DH), lambda i: (0, i, 0)),
            pl.BlockSpec((D, D), lambda i: (0, 0)),
            pl.BlockSpec((D, D), lambda i: (0, 0)),
            pl.BlockSpec((1, D), lambda i: (0, 0)),
        ],
        out_specs=out_specs,
        out_shape=out_shape,
        name="tc_layer_relu" if relu else "tc_layer",
    )


_tc_layer_relu = _tc_layer(True, True)
_tc_layer_lin = _tc_layer(False, False)


def kernel(x, edge_index, W1l, b1l, W1r, b1r, W2l, b2l, W2r, b2r):
    src = edge_index[0]
    dst = edge_index[1]
    pad = EPAD - E
    srcs = jnp.concatenate([src, jnp.zeros((pad,), jnp.int32)]).reshape(NS, NCHUNK, CH)
    dsts = jnp.concatenate([dst, jnp.full((pad,), N, jnp.int32)]).reshape(NS, NCHUNK, CH)
    edges = jnp.stack([srcs, dsts], axis=1)

    x2 = jnp.stack([x[:, :DH], x[:, DH:]])
    w1l_t = W1l.T
    w1r_t = W1r.T
    b1 = (b1l + b1r).reshape(1, D)
    w2l_t = jnp.zeros((D, D), jnp.float32).at[:, :OUT].set(W2l.T)
    w2r_t = jnp.zeros((D, D), jnp.float32).at[:, :OUT].set(W2r.T)
    b2 = jnp.zeros((1, D), jnp.float32).at[0, :OUT].set(b2l + b2r)

    sums1, cnt = _sc_agg_cnt_kernel(x2, edges)
    cnt3 = cnt.reshape(NC, NP, 1)
    h2 = _tc_layer_relu(sums1, cnt3, x2, w1l_t, w1r_t, b1)
    (sums2,) = _sc_agg_kernel(h2, edges)
    return _tc_layer_lin(sums2, cnt3, h2, w2l_t, w2r_t, b2)


# speedup vs baseline: 1.0963x; 1.0758x over previous
import jax
import jax.numpy as jnp
from jax import lax
from jax.experimental import pallas as pl
from jax.experimental.pallas import tpu as pltpu
from jax.experimental.pallas import tpu_sc as plsc

N = 10000
E = 320000
D = 128
DH = D // 2
OUT = 121

NC, NS = 2, 16
CH = 128
NP = 10240
STRIPE = NP // NS
KBUF = 2
NCHUNK = KBUF * (-(-E // (NS * CH * KBUF)))
EPAD = NS * CH * NCHUNK


def _sc_agg(with_cnt):
    mesh = plsc.VectorSubcoreMesh(core_axis_name="c", subcore_axis_name="s")
    out_type = [jax.ShapeDtypeStruct((NC, NP, DH), jnp.float32)]
    scratch = [
        pltpu.VMEM((2, NCHUNK, CH), jnp.int32),
    ] + [pltpu.VMEM((CH, DH), jnp.float32) for _ in range(KBUF)] + [
        pltpu.VMEM((CH, DH), jnp.float32),
        pltpu.VMEM_SHARED((NP, DH), jnp.float32),
    ] + [pltpu.SemaphoreType.DMA for _ in range(KBUF)]
    if with_cnt:
        out_type.append(jax.ShapeDtypeStruct((NC, NP), jnp.float32))
        scratch += [
            pltpu.VMEM((CH,), jnp.float32),
            pltpu.VMEM((STRIPE,), jnp.float32),
            pltpu.VMEM_SHARED((NP,), jnp.float32),
            pltpu.SemaphoreType.DMA,
        ]

    def body(x2_hbm, edges_hbm, *rest):
        if with_cnt:
            sums_out, cnt_out = rest[0], rest[1]
            rest = rest[2:]
        else:
            sums_out = rest[0]
            rest = rest[1:]
        idx_v = rest[0]
        bufs = rest[1:1 + KBUF]
        zero_v = rest[1 + KBUF]
        sums_sh = rest[2 + KBUF]
        gsems = rest[3 + KBUF:3 + 2 * KBUF]
        if with_cnt:
            ones_v, zcnt_v, cnt_sh, csem = rest[3 + 2 * KBUF:]
        cid = lax.axis_index("c")
        sid = lax.axis_index("s")

        pltpu.sync_copy(edges_hbm.at[sid], idx_v)
        src_v = idx_v.at[0]
        dst_v = idx_v.at[1]

        @pl.loop(0, CH)
        def _zero_rows(i):
            for j in range(DH // 16):
                zero_v[i, pl.ds(j * 16, 16)] = jnp.zeros((16,), jnp.float32)

        for k in range(STRIPE // CH):
            pltpu.sync_copy(zero_v, sums_sh.at[pl.ds(sid * STRIPE + k * CH, CH)])
        if with_cnt:
            @pl.loop(0, STRIPE // 16)
            def _zero_cnt(i):
                zcnt_v[pl.ds(i * 16, 16)] = jnp.zeros((16,), jnp.float32)

            @pl.loop(0, CH // 16)
            def _ones(i):
                ones_v[pl.ds(i * 16, 16)] = jnp.ones((16,), jnp.float32)

            pltpu.sync_copy(zcnt_v, cnt_sh.at[pl.ds(sid * STRIPE, STRIPE)])
        plsc.subcore_barrier()

        cnt_lo = cid * (NCHUNK // 2)
        cnt_hi = cnt_lo + NCHUNK // 2

        def _maybe_cnt(j):
            if with_cnt:
                @pl.when((j >= cnt_lo) & (j < cnt_hi))
                def _():
                    pltpu.async_copy(ones_v, cnt_sh.at[dst_v.at[j]], csem,
                                     add=True)

        for b in range(KBUF):
            pltpu.async_copy(x2_hbm.at[cid].at[src_v.at[b]], bufs[b], gsems[b])

        @pl.loop(0, NCHUNK, step=KBUF)
        def _chunk(j):
            for b in range(KBUF):
                jj = j + b
                pltpu.make_async_copy(x2_hbm.at[cid].at[src_v.at[jj]],
                                      bufs[b], gsems[b]).wait()
                pltpu.sync_copy(bufs[b], sums_sh.at[dst_v.at[jj]], add=True)
                _maybe_cnt(jj)

                @pl.when(jj + KBUF < NCHUNK)
                def _():
                    pltpu.async_copy(x2_hbm.at[cid].at[src_v.at[jj + KBUF]],
                                     bufs[b], gsems[b])

        if with_cnt:
            @pl.loop(cnt_lo, cnt_hi)
            def _drain_cnt(j):
                pltpu.make_async_copy(ones_v, cnt_sh.at[dst_v.at[j]],
                                      csem).wait()
        plsc.subcore_barrier()
        pltpu.sync_copy(sums_sh.at[pl.ds(sid * STRIPE, STRIPE)],
                        sums_out.at[cid, pl.ds(sid * STRIPE, STRIPE)])
        if with_cnt:
            pltpu.sync_copy(cnt_sh.at[pl.ds(sid * STRIPE, STRIPE)],
                            cnt_out.at[cid, pl.ds(sid * STRIPE, STRIPE)])

    return pl.kernel(body, out_type=out_type, mesh=mesh, scratch_types=scratch,
                     compiler_params=pltpu.CompilerParams(use_tc_tiling_on_sc=False),
                     name="sc_agg_cnt" if with_cnt else "sc_agg")


_sc_agg_cnt_kernel = _sc_agg(True)
_sc_agg_kernel = _sc_agg(False)


def _tc_layer(relu, split_out):
    TR = 1024 if split_out else 1000

    def body(sums_ref, cnt_ref, x_ref, wl_ref, wr_ref, b_ref, o_ref):
        c = cnt_ref[0] + cnt_ref[1]
        s = jnp.concatenate([sums_ref[0], sums_ref[1]], axis=1)
        mean = s * (1.0 / jnp.maximum(c, 1.0))
        xcat = jnp.concatenate([x_ref[0], x_ref[1]], axis=1)
        acc = jnp.dot(mean, wl_ref[...], preferred_element_type=jnp.float32)
        acc = acc + jnp.dot(xcat, wr_ref[...],
                            preferred_element_type=jnp.float32)
        acc = acc + b_ref[...]
        if relu:
            acc = jnp.maximum(acc, 0.0)
        if split_out:
            o_ref[0] = acc[:, :DH]
            o_ref[1] = acc[:, DH:]
        else:
            o_ref[...] = acc[:, :OUT]

    if split_out:
        out_specs = pl.BlockSpec((NC, TR, DH), lambda i: (0, i, 0))
        out_shape = jax.ShapeDtypeStruct((NC, NP, DH), jnp.float32)
    else:
        out_specs = pl.BlockSpec((TR, OUT), lambda i: (i, 0))
        out_shape = jax.ShapeDtypeStruct((N, OUT), jnp.float32)
    return pl.pallas_call(
        body,
        grid=((NP // TR) if split_out else (N // TR),),
        in_specs=[
            pl.BlockSpec((NC, TR, DH), lambda i: (0, i, 0)),
            pl.BlockSpec((NC, TR, 1), lambda i: (0, i, 0)),
            pl.BlockSpec((NC, TR, DH), lambda i: (0, i, 0)),
            pl.BlockSpec((D, D), lambda i: (0, 0)),
            pl.BlockSpec((D, D), lambda i: (0, 0)),
            pl.BlockSpec((1, D), lambda i: (0, 0)),
        ],
        out_specs=out_specs,
        out_shape=out_shape,
        name="tc_layer_relu" if relu else "tc_layer",
    )


_tc_layer_relu = _tc_layer(True, True)
_tc_layer_lin = _tc_layer(False, False)


def kernel(x, edge_index, W1l, b1l, W1r, b1r, W2l, b2l, W2r, b2r):
    src = edge_index[0]
    dst = edge_index[1]
    pad = EPAD - E
    srcs = jnp.concatenate([src, jnp.zeros((pad,), jnp.int32)]).reshape(NS, NCHUNK, CH)
    dsts = jnp.concatenate([dst, jnp.full((pad,), N, jnp.int32)]).reshape(NS, NCHUNK, CH)
    edges = jnp.stack([srcs, dsts], axis=1)

    xp = jnp.zeros((NP, D), jnp.float32).at[:N].set(x)
    x2 = jnp.stack([xp[:, :DH], xp[:, DH:]])
    w1l_t = W1l.T
    w1r_t = W1r.T
    b1 = (b1l + b1r).reshape(1, D)
    w2l_t = jnp.zeros((D, D), jnp.float32).at[:, :OUT].set(W2l.T)
    w2r_t = jnp.zeros((D, D), jnp.float32).at[:, :OUT].set(W2r.T)
    b2 = jnp.zeros((1, D), jnp.float32).at[0, :OUT].set(b2l + b2r)

    sums1, cnt = _sc_agg_cnt_kernel(x2, edges)
    cnt3 = cnt.reshape(NC, NP, 1)
    h2 = _tc_layer_relu(sums1, cnt3, x2, w1l_t, w1r_t, b1)
    (sums2,) = _sc_agg_kernel(h2, edges)
    return _tc_layer_lin(sums2, cnt3, h2, w2l_t, w2r_t, b2)
